# baseline (speedup 1.0000x reference)
import jax
import jax.numpy as jnp
from jax import lax
from jax.experimental import pallas as pl
from jax.experimental.pallas import tpu as pltpu

SIZE_ROWS = 16384
OBS_D = 1024
BLK = 256
NBLK = SIZE_ROWS // BLK
VR, VC = 128, 128


def _shift_flat(x, newval):
    r, c = x.shape
    within = pltpu.roll(x, 1, 1)
    col = pltpu.roll(x[:, c - 1:c], 1, 0)
    ridx = lax.broadcasted_iota(jnp.int32, (r, c), 0)
    cidx = lax.broadcasted_iota(jnp.int32, (r, c), 1)
    out = jnp.where(cidx == 0, jnp.broadcast_to(col, (r, c)), within)
    return jnp.where((ridx == 0) & (cidx == 0), newval, out)


def _body(cur_ref, prev_ref, obs_new_ref, act_ref, rew_ref, scal_ref,
          obs_out_ref, act_out_ref, rew_out_ref):
    i = pl.program_id(0)
    cur = cur_ref[...]
    rolled = pltpu.roll(cur, 1, 0)
    row0 = jnp.where(i == 0, obs_new_ref[0, :], prev_ref[7, :])
    ridx = lax.broadcasted_iota(jnp.int32, (BLK, OBS_D), 0)
    obs_out_ref[...] = jnp.where(ridx == 0, row0[None, :], rolled)

    @pl.when(i == 0)
    def _small():
        act_out_ref[...] = _shift_flat(act_ref[...], scal_ref[0, 0])
        rew_out_ref[...] = _shift_flat(rew_ref[...], scal_ref[0, 1])


def kernel(buffer_observations, buffer_actions, buffer_rewards,
           observation, action, reward):
    act2d = buffer_actions.reshape(VR, VC)
    rew2d = buffer_rewards.reshape(VR, VC)
    obs_new = observation.reshape(1, OBS_D)
    scal = jnp.stack([action, reward]).reshape(1, 2)

    grid = (NBLK,)
    obs_out, act_out, rew_out = pl.pallas_call(
        _body,
        grid=grid,
        in_specs=[
            pl.BlockSpec((BLK, OBS_D), lambda i: (i, 0)),
            pl.BlockSpec((8, OBS_D), lambda i: (jnp.maximum(i * (BLK // 8) - 1, 0), 0)),
            pl.BlockSpec((1, OBS_D), lambda i: (0, 0)),
            pl.BlockSpec((VR, VC), lambda i: (0, 0)),
            pl.BlockSpec((VR, VC), lambda i: (0, 0)),
            pl.BlockSpec(memory_space=pltpu.SMEM),
        ],
        out_specs=[
            pl.BlockSpec((BLK, OBS_D), lambda i: (i, 0)),
            pl.BlockSpec((VR, VC), lambda i: (0, 0)),
            pl.BlockSpec((VR, VC), lambda i: (0, 0)),
        ],
        out_shape=[
            jax.ShapeDtypeStruct((SIZE_ROWS, OBS_D), jnp.float32),
            jax.ShapeDtypeStruct((VR, VC), jnp.float32),
            jax.ShapeDtypeStruct((VR, VC), jnp.float32),
        ],
        compiler_params=pltpu.CompilerParams(
            dimension_semantics=("arbitrary",),
        ),
    )(buffer_observations, buffer_observations, obs_new, act2d, rew2d, scal)

    return (obs_out, act_out.reshape(SIZE_ROWS), rew_out.reshape(SIZE_ROWS))


# speedup vs baseline: 1.3727x; 1.3727x over previous
import jax
import jax.numpy as jnp
from jax import lax
from jax.experimental import pallas as pl
from jax.experimental.pallas import tpu as pltpu

SIZE_ROWS = 16384
OBS_D = 1024
BLK = 512
NBLK = SIZE_ROWS // BLK
VR, VC = 128, 128


def _shift_flat(x, newval):
    r, c = x.shape
    within = pltpu.roll(x, 1, 1)
    col = pltpu.roll(x[:, c - 1:c], 1, 0)
    ridx = lax.broadcasted_iota(jnp.int32, (r, c), 0)
    cidx = lax.broadcasted_iota(jnp.int32, (r, c), 1)
    out = jnp.where(cidx == 0, jnp.broadcast_to(col, (r, c)), within)
    return jnp.where((ridx == 0) & (cidx == 0), newval, out)


def _body(cur_ref, prev_ref, obs_new_ref, act_ref, rew_ref, scal_ref,
          obs_out_ref, act_out_ref, rew_out_ref):
    i = pl.program_id(0)
    obs_out_ref[...] = pltpu.roll(cur_ref[...], 1, 0)
    row0 = jnp.where(i == 0, obs_new_ref[0, :], prev_ref[7, :])
    obs_out_ref[0:1, :] = row0[None, :]

    @pl.when(i == 0)
    def _small():
        act_out_ref[...] = _shift_flat(act_ref[...], scal_ref[0, 0])
        rew_out_ref[...] = _shift_flat(rew_ref[...], scal_ref[0, 1])


def kernel(buffer_observations, buffer_actions, buffer_rewards,
           observation, action, reward):
    act2d = buffer_actions.reshape(VR, VC)
    rew2d = buffer_rewards.reshape(VR, VC)
    obs_new = observation.reshape(1, OBS_D)
    scal = jnp.stack([action, reward]).reshape(1, 2)

    grid = (NBLK,)
    obs_out, act_out, rew_out = pl.pallas_call(
        _body,
        grid=grid,
        in_specs=[
            pl.BlockSpec((BLK, OBS_D), lambda i: (i, 0)),
            pl.BlockSpec((8, OBS_D), lambda i: (jnp.maximum(i * (BLK // 8) - 1, 0), 0)),
            pl.BlockSpec((1, OBS_D), lambda i: (0, 0)),
            pl.BlockSpec((VR, VC), lambda i: (0, 0)),
            pl.BlockSpec((VR, VC), lambda i: (0, 0)),
            pl.BlockSpec(memory_space=pltpu.SMEM),
        ],
        out_specs=[
            pl.BlockSpec((BLK, OBS_D), lambda i: (i, 0)),
            pl.BlockSpec((VR, VC), lambda i: (0, 0)),
            pl.BlockSpec((VR, VC), lambda i: (0, 0)),
        ],
        out_shape=[
            jax.ShapeDtypeStruct((SIZE_ROWS, OBS_D), jnp.float32),
            jax.ShapeDtypeStruct((VR, VC), jnp.float32),
            jax.ShapeDtypeStruct((VR, VC), jnp.float32),
        ],
        compiler_params=pltpu.CompilerParams(
            dimension_semantics=("arbitrary",),
        ),
    )(buffer_observations, buffer_observations, obs_new, act2d, rew2d, scal)

    return (obs_out, act_out.reshape(SIZE_ROWS), rew_out.reshape(SIZE_ROWS))


# speedup vs baseline: 1.4931x; 1.0877x over previous
import jax
import jax.numpy as jnp
from jax import lax
from jax.experimental import pallas as pl
from jax.experimental.pallas import tpu as pltpu

SIZE_ROWS = 16384
OBS_D = 1024
BLK = 1024
NBLK = SIZE_ROWS // BLK
VR, VC = 128, 128


def _shift_flat(x, newval):
    r, c = x.shape
    within = pltpu.roll(x, 1, 1)
    col = pltpu.roll(x[:, c - 1:c], 1, 0)
    ridx = lax.broadcasted_iota(jnp.int32, (r, c), 0)
    cidx = lax.broadcasted_iota(jnp.int32, (r, c), 1)
    out = jnp.where(cidx == 0, jnp.broadcast_to(col, (r, c)), within)
    return jnp.where((ridx == 0) & (cidx == 0), newval, out)


def _body(cur_ref, prev_ref, obs_new_ref, act_ref, rew_ref, scal_ref,
          obs_out_ref, act_out_ref, rew_out_ref):
    i = pl.program_id(0)
    obs_out_ref[...] = pltpu.roll(cur_ref[...], 1, 0)
    row0 = jnp.where(i == 0, obs_new_ref[0, :], prev_ref[7, :])
    obs_out_ref[0:1, :] = row0[None, :]

    @pl.when(i == 0)
    def _small():
        act_out_ref[...] = _shift_flat(act_ref[...], scal_ref[0, 0])
        rew_out_ref[...] = _shift_flat(rew_ref[...], scal_ref[0, 1])


def kernel(buffer_observations, buffer_actions, buffer_rewards,
           observation, action, reward):
    act2d = buffer_actions.reshape(VR, VC)
    rew2d = buffer_rewards.reshape(VR, VC)
    obs_new = observation.reshape(1, OBS_D)
    scal = jnp.stack([action, reward]).reshape(1, 2)

    grid = (NBLK,)
    obs_out, act_out, rew_out = pl.pallas_call(
        _body,
        grid=grid,
        in_specs=[
            pl.BlockSpec((BLK, OBS_D), lambda i: (i, 0)),
            pl.BlockSpec((8, OBS_D), lambda i: (jnp.maximum(i * (BLK // 8) - 1, 0), 0)),
            pl.BlockSpec((1, OBS_D), lambda i: (0, 0)),
            pl.BlockSpec((VR, VC), lambda i: (0, 0)),
            pl.BlockSpec((VR, VC), lambda i: (0, 0)),
            pl.BlockSpec(memory_space=pltpu.SMEM),
        ],
        out_specs=[
            pl.BlockSpec((BLK, OBS_D), lambda i: (i, 0)),
            pl.BlockSpec((VR, VC), lambda i: (0, 0)),
            pl.BlockSpec((VR, VC), lambda i: (0, 0)),
        ],
        out_shape=[
            jax.ShapeDtypeStruct((SIZE_ROWS, OBS_D), jnp.float32),
            jax.ShapeDtypeStruct((VR, VC), jnp.float32),
            jax.ShapeDtypeStruct((VR, VC), jnp.float32),
        ],
        compiler_params=pltpu.CompilerParams(
            dimension_semantics=("arbitrary",),
        ),
    )(buffer_observations, buffer_observations, obs_new, act2d, rew2d, scal)

    return (obs_out, act_out.reshape(SIZE_ROWS), rew_out.reshape(SIZE_ROWS))


# speedup vs baseline: 1.5597x; 1.0446x over previous
import jax
import jax.numpy as jnp
from jax import lax
from jax.experimental import pallas as pl
from jax.experimental.pallas import tpu as pltpu

SIZE_ROWS = 16384
OBS_D = 1024
BLK = 2048
NBLK = SIZE_ROWS // BLK
VR, VC = 128, 128


def _shift_flat(x, newval):
    r, c = x.shape
    within = pltpu.roll(x, 1, 1)
    col = pltpu.roll(x[:, c - 1:c], 1, 0)
    ridx = lax.broadcasted_iota(jnp.int32, (r, c), 0)
    cidx = lax.broadcasted_iota(jnp.int32, (r, c), 1)
    out = jnp.where(cidx == 0, jnp.broadcast_to(col, (r, c)), within)
    return jnp.where((ridx == 0) & (cidx == 0), newval, out)


def _body(cur_ref, prev_ref, obs_new_ref, act_ref, rew_ref, scal_ref,
          obs_out_ref, act_out_ref, rew_out_ref):
    i = pl.program_id(0)
    obs_out_ref[...] = pltpu.roll(cur_ref[...], 1, 0)
    row0 = jnp.where(i == 0, obs_new_ref[0, :], prev_ref[7, :])
    obs_out_ref[0:1, :] = row0[None, :]

    @pl.when(i == 0)
    def _small():
        act_out_ref[...] = _shift_flat(act_ref[...], scal_ref[0, 0])
        rew_out_ref[...] = _shift_flat(rew_ref[...], scal_ref[0, 1])


def kernel(buffer_observations, buffer_actions, buffer_rewards,
           observation, action, reward):
    act2d = buffer_actions.reshape(VR, VC)
    rew2d = buffer_rewards.reshape(VR, VC)
    obs_new = observation.reshape(1, OBS_D)
    scal = jnp.stack([action, reward]).reshape(1, 2)

    grid = (NBLK,)
    obs_out, act_out, rew_out = pl.pallas_call(
        _body,
        grid=grid,
        in_specs=[
            pl.BlockSpec((BLK, OBS_D), lambda i: (i, 0)),
            pl.BlockSpec((8, OBS_D), lambda i: (jnp.maximum(i * (BLK // 8) - 1, 0), 0)),
            pl.BlockSpec((1, OBS_D), lambda i: (0, 0)),
            pl.BlockSpec((VR, VC), lambda i: (0, 0)),
            pl.BlockSpec((VR, VC), lambda i: (0, 0)),
            pl.BlockSpec(memory_space=pltpu.SMEM),
        ],
        out_specs=[
            pl.BlockSpec((BLK, OBS_D), lambda i: (i, 0)),
            pl.BlockSpec((VR, VC), lambda i: (0, 0)),
            pl.BlockSpec((VR, VC), lambda i: (0, 0)),
        ],
        out_shape=[
            jax.ShapeDtypeStruct((SIZE_ROWS, OBS_D), jnp.float32),
            jax.ShapeDtypeStruct((VR, VC), jnp.float32),
            jax.ShapeDtypeStruct((VR, VC), jnp.float32),
        ],
        compiler_params=pltpu.CompilerParams(
            dimension_semantics=("arbitrary",),
        ),
    )(buffer_observations, buffer_observations, obs_new, act2d, rew2d, scal)

    return (obs_out, act_out.reshape(SIZE_ROWS), rew_out.reshape(SIZE_ROWS))
